# baseline (speedup 1.0000x reference)
import jax
import jax.numpy as jnp
from jax.experimental import pallas as pl
from jax.experimental.pallas import tpu as pltpu
from jax.experimental.pallas import tpu_sc as plsc

B, N, DIM = 16, 1024, 768
CB, CD = 8192, 256
TOK = B * N
TT = 256
CT = 2048
NC = CB // CT
LN_EPS = 1e-5
COMMIT = 0.25

_BF = jnp.bfloat16
_NT = (((1,), (1,)), ((), ()))
_NN = (((1,), (0,)), ((), ()))


def _vq_main_kernel(h_ref, w1_ref, b1_ref, embb_ref,
                    esq_ref, idx_ref, mind_ref):
    flat = jax.lax.dot_general(h_ref[...], w1_ref[...],
                               _NN, preferred_element_type=jnp.float32)
    flat = flat + b1_ref[...]
    zsq = jnp.sum(flat * flat, axis=1, keepdims=True)
    flatb = flat.astype(_BF)

    rmin = None
    ridx = None
    for c in range(NC):
        em = embb_ref[c * CT:(c + 1) * CT, :]
        dot = jax.lax.dot_general(flatb, em, _NT,
                                  preferred_element_type=jnp.float32)
        dist = zsq - 2.0 * dot + esq_ref[:, c * CT:(c + 1) * CT]
        tmin = jnp.min(dist, axis=1, keepdims=True)
        targ = jnp.argmin(dist, axis=1, keepdims=True)
        targ = targ.astype(jnp.int32) + jnp.int32(c * CT)
        if c == 0:
            rmin, ridx = tmin, targ
        else:
            upd = tmin < rmin
            rmin = jnp.where(upd, tmin, rmin)
            ridx = jnp.where(upd, targ, ridx)

    idx_ref[...] = ridx
    mind_ref[...] = rmin


def _embw2_kernel(embed_ref, w2_ref, b2_ref, ews_ref, esq_ref):
    em = embed_ref[...]
    esq_ref[...] = jnp.sum(em * em, axis=1, keepdims=True)
    ew = jax.lax.dot_general(em.astype(_BF), w2_ref[...].astype(_BF),
                             _NN, preferred_element_type=jnp.float32)
    ew = ew + b2_ref[...]
    ews_ref[0] = ew[:, :DIM // 2]
    ews_ref[1] = ew[:, DIM // 2:]


_CPT = 1024
_GW = 128


def _sc_gather(ews, idx_row):
    @pl.kernel(out_type=jax.ShapeDtypeStruct((TOK, DIM), jnp.float32),
               mesh=plsc.VectorSubcoreMesh(core_axis_name="c",
                                           subcore_axis_name="s"))
    def _k(ews_hbm, i_hbm, o_hbm):
        for j in range(2):
            src = ews_hbm.at[j]

            def body(i_vmem, o_vmem, src=src):
                pltpu.sync_copy(src.at[i_vmem.at[0]], o_vmem)

            pltpu.emit_pipeline(
                body,
                grid=(TOK // _GW,),
                in_specs=[pl.BlockSpec((1, _GW), lambda i: (0, i))],
                out_specs=[pl.BlockSpec((_GW, DIM // 2),
                                        lambda i, j=j: (i, j))],
                core_axis_name=("c", "s"),
                dimension_semantics=(pltpu.PARALLEL,),
            )(i_hbm, o_hbm)

    return _k(ews, idx_row)


def kernel(x, ln_gamma, ln_beta, W1, b1, embed, W2, b2):
    xf = x.reshape(TOK, DIM).astype(jnp.float32)
    mu = jnp.mean(xf, axis=-1, keepdims=True)
    var = jnp.mean((xf - mu) ** 2, axis=-1, keepdims=True)
    x_normed = (xf - mu) / jnp.sqrt(var + LN_EPS) * ln_gamma + ln_beta
    h = jax.nn.gelu(x_normed, approximate=False).astype(_BF)
    embb = embed.astype(_BF)
    w1b = W1.astype(_BF)

    ews, esq2 = pl.pallas_call(
        _embw2_kernel,
        grid=(CB // _CPT,),
        in_specs=[
            pl.BlockSpec((_CPT, CD), lambda i: (i, 0)),
            pl.BlockSpec((CD, DIM), lambda i: (0, 0)),
            pl.BlockSpec((DIM,), lambda i: (0,)),
        ],
        out_specs=[
            pl.BlockSpec((2, _CPT, DIM // 2), lambda i: (0, i, 0)),
            pl.BlockSpec((_CPT, 1), lambda i: (i, 0)),
        ],
        out_shape=[
            jax.ShapeDtypeStruct((2, CB, DIM // 2), jnp.float32),
            jax.ShapeDtypeStruct((CB, 1), jnp.float32),
        ],
    )(embed, W2, b2)

    esq_row = esq2.reshape(1, CB)

    idx, mind = pl.pallas_call(
        _vq_main_kernel,
        grid=(TOK // TT,),
        in_specs=[
            pl.BlockSpec((TT, DIM), lambda i: (i, 0)),
            pl.BlockSpec((DIM, CD), lambda i: (0, 0)),
            pl.BlockSpec((CD,), lambda i: (0,)),
            pl.BlockSpec((CB, CD), lambda i: (0, 0)),
            pl.BlockSpec((1, CB), lambda i: (0, 0)),
        ],
        out_specs=[
            pl.BlockSpec((TT, 1), lambda i: (i, 0)),
            pl.BlockSpec((TT, 1), lambda i: (i, 0)),
        ],
        out_shape=[
            jax.ShapeDtypeStruct((TOK, 1), jnp.int32),
            jax.ShapeDtypeStruct((TOK, 1), jnp.float32),
        ],
    )(h, w1b, b1, embb, esq_row)

    indices = idx.reshape(B, N)
    quantized = _sc_gather(ews, idx.reshape(1, TOK)).reshape(B, N, DIM)
    commitment_loss = COMMIT * (jnp.sum(mind) / (TOK * CD))
    return quantized, indices, commitment_loss


# speedup vs baseline: 1.2082x; 1.2082x over previous
import functools

import jax
import jax.numpy as jnp
from jax.experimental import pallas as pl
from jax.experimental.pallas import tpu as pltpu
from jax.experimental.pallas import tpu_sc as plsc

B, N, DIM = 16, 1024, 768
CB, CD = 8192, 256
TOK = B * N
TT = 256
CT = 2048
NC = CB // CT
LN_EPS = 1e-5
COMMIT = 0.25

_BF = jnp.bfloat16
_NT = (((1,), (1,)), ((), ()))
_NN = (((1,), (0,)), ((), ()))


def _vq_main_kernel(h_ref, w1_ref, b1_ref, embb_ref,
                    esq_ref, idx_ref, mind_ref):
    flat = jax.lax.dot_general(h_ref[...], w1_ref[...],
                               _NN, preferred_element_type=jnp.float32)
    flat = flat + b1_ref[...]
    zsq = jnp.sum(flat * flat, axis=1, keepdims=True)
    flatb = flat.astype(_BF)

    rmin = None
    ridx = None
    for c in range(NC):
        em = embb_ref[c * CT:(c + 1) * CT, :]
        dot = jax.lax.dot_general(flatb, em, _NT,
                                  preferred_element_type=jnp.float32)
        dist = zsq - 2.0 * dot + esq_ref[:, c * CT:(c + 1) * CT]
        tmin = jnp.min(dist, axis=1, keepdims=True)
        targ = jnp.argmin(dist, axis=1, keepdims=True)
        targ = targ.astype(jnp.int32) + jnp.int32(c * CT)
        if c == 0:
            rmin, ridx = tmin, targ
        else:
            upd = tmin < rmin
            rmin = jnp.where(upd, tmin, rmin)
            ridx = jnp.where(upd, targ, ridx)

    idx_ref[...] = ridx
    mind_ref[...] = rmin


def _embw2_kernel(embed_ref, w2_ref, b2_ref, ew_ref, esq_ref):
    em = embed_ref[...]
    esq_ref[...] = jnp.sum(em * em, axis=1, keepdims=True)
    ew = jax.lax.dot_general(em.astype(_BF), w2_ref[...].astype(_BF),
                             _NN, preferred_element_type=jnp.float32)
    ew_ref[...] = ew + b2_ref[...]


_CPT = 1024
_GW = 128


_NWORK = 32
_BPW = TOK // _NWORK
_GCH = 128


def _sc_gather(ew, idx_flat):
    mesh = plsc.VectorSubcoreMesh(core_axis_name="c", subcore_axis_name="s")

    @functools.partial(
        pl.kernel, mesh=mesh,
        out_type=jax.ShapeDtypeStruct((TOK, DIM), jnp.float32),
        scratch_types=[
            pltpu.VMEM((_GCH,), jnp.int32),
            pltpu.VMEM((_GCH, DIM), jnp.float32),
            pltpu.SemaphoreType.DMA,
        ],
    )
    def _k(ew_hbm, idx_hbm, out_hbm, idx_v, rows_v, sem):
        wid = jax.lax.axis_index("s") * 2 + jax.lax.axis_index("c")
        base = wid * _BPW

        @pl.loop(0, _BPW // _GCH)
        def _(j):
            off = base + j * _GCH
            pltpu.sync_copy(idx_hbm.at[pl.ds(off, _GCH)], idx_v)
            pltpu.async_copy(ew_hbm.at[idx_v], rows_v, sem).wait()
            pltpu.sync_copy(rows_v, out_hbm.at[pl.ds(off, _GCH)])

    return _k(ew, idx_flat)


def kernel(x, ln_gamma, ln_beta, W1, b1, embed, W2, b2):
    xf = x.reshape(TOK, DIM).astype(jnp.float32)
    mu = jnp.mean(xf, axis=-1, keepdims=True)
    var = jnp.mean((xf - mu) ** 2, axis=-1, keepdims=True)
    x_normed = (xf - mu) / jnp.sqrt(var + LN_EPS) * ln_gamma + ln_beta
    h = jax.nn.gelu(x_normed, approximate=False).astype(_BF)
    embb = embed.astype(_BF)
    w1b = W1.astype(_BF)

    ew, esq2 = pl.pallas_call(
        _embw2_kernel,
        grid=(CB // _CPT,),
        in_specs=[
            pl.BlockSpec((_CPT, CD), lambda i: (i, 0)),
            pl.BlockSpec((CD, DIM), lambda i: (0, 0)),
            pl.BlockSpec((DIM,), lambda i: (0,)),
        ],
        out_specs=[
            pl.BlockSpec((_CPT, DIM), lambda i: (i, 0)),
            pl.BlockSpec((_CPT, 1), lambda i: (i, 0)),
        ],
        out_shape=[
            jax.ShapeDtypeStruct((CB, DIM), jnp.float32),
            jax.ShapeDtypeStruct((CB, 1), jnp.float32),
        ],
    )(embed, W2, b2)

    esq_row = esq2.reshape(1, CB)

    idx, mind = pl.pallas_call(
        _vq_main_kernel,
        grid=(TOK // TT,),
        in_specs=[
            pl.BlockSpec((TT, DIM), lambda i: (i, 0)),
            pl.BlockSpec((DIM, CD), lambda i: (0, 0)),
            pl.BlockSpec((CD,), lambda i: (0,)),
            pl.BlockSpec((CB, CD), lambda i: (0, 0)),
            pl.BlockSpec((1, CB), lambda i: (0, 0)),
        ],
        out_specs=[
            pl.BlockSpec((TT, 1), lambda i: (i, 0)),
            pl.BlockSpec((TT, 1), lambda i: (i, 0)),
        ],
        out_shape=[
            jax.ShapeDtypeStruct((TOK, 1), jnp.int32),
            jax.ShapeDtypeStruct((TOK, 1), jnp.float32),
        ],
    )(h, w1b, b1, embb, esq_row)

    indices = idx.reshape(B, N)
    quantized = _sc_gather(ew, idx.reshape(TOK)).reshape(B, N, DIM)
    commitment_loss = COMMIT * (jnp.sum(mind) / (TOK * CD))
    return quantized, indices, commitment_loss
